# Initial kernel scaffold; baseline (speedup 1.0000x reference)
import jax
import jax.numpy as jnp
from jax.experimental import pallas as pl


def kernel(node_emb, edge_index, etypes, basis1, wcomp1, loop1, bias1, basis2, wcomp2, loop2, bias2):
    raise NotImplementedError("write your pallas kernel here")


# speedup vs baseline: 9.0423x; 9.0423x over previous
import functools

import jax
import jax.numpy as jnp
from jax import lax
from jax.experimental import pallas as pl
from jax.experimental.pallas import tpu as pltpu
from jax.experimental.pallas import tpu_sc as plsc

N_NODES = 10000
N_EDGES = 320000
NUM_REL = 8
DIM = 128

NC = 2
NS = 16
NW = NC * NS

E_PAD = 327680
IDX_ROWS = E_PAD // DIM
ROWS_PER_W = IDX_ROWS // NW
AGG_ROWS = 10240
ROWS_PER_TILE = AGG_ROWS // NS

N_BLK = 1000
N_GRID = N_NODES // N_BLK


def _dense_pre_body(h_ref, basis_ref, wcomp_ref, loop_ref, bias_ref,
                    hw_ref, hsl_ref):
  r = pl.program_id(1)
  wc = wcomp_ref[pl.ds(r, 1), :]
  w = wc[:, 0:1] * basis_ref[0] + wc[:, 1:2] * basis_ref[1]
  hw_ref[0] = jnp.dot(h_ref[...], w, preferred_element_type=jnp.float32)

  @pl.when(r == 0)
  def _():
    hsl_ref[...] = (
        jnp.dot(h_ref[...], loop_ref[...], preferred_element_type=jnp.float32)
        + bias_ref[...])


def _dense_pre(h, basis, wcomp, loop_w, bias2d):
  return pl.pallas_call(
      _dense_pre_body,
      grid=(N_GRID, NUM_REL),
      in_specs=[
          pl.BlockSpec((N_BLK, DIM), lambda n, r: (n, 0)),
          pl.BlockSpec((2, DIM, DIM), lambda n, r: (0, 0, 0)),
          pl.BlockSpec((NUM_REL, 2), lambda n, r: (0, 0)),
          pl.BlockSpec((DIM, DIM), lambda n, r: (0, 0)),
          pl.BlockSpec((1, DIM), lambda n, r: (0, 0)),
      ],
      out_specs=[
          pl.BlockSpec((1, N_BLK, DIM), lambda n, r: (r, n, 0)),
          pl.BlockSpec((N_BLK, DIM), lambda n, r: (n, 0)),
      ],
      out_shape=[
          jax.ShapeDtypeStruct((NUM_REL, N_NODES, DIM), jnp.float32),
          jax.ShapeDtypeStruct((N_NODES, DIM), jnp.float32),
      ],
  )(h, basis, wcomp, loop_w, bias2d)


def _gidx_body(src_ref, et_ref, out_ref):
  out_ref[...] = et_ref[...] * N_NODES + src_ref[...]


def _gidx(src2d, et2d):
  blk = IDX_ROWS // 10
  return pl.pallas_call(
      _gidx_body,
      grid=(10,),
      in_specs=[
          pl.BlockSpec((blk, DIM), lambda i: (i, 0)),
          pl.BlockSpec((blk, DIM), lambda i: (i, 0)),
      ],
      out_specs=pl.BlockSpec((blk, DIM), lambda i: (i, 0)),
      out_shape=jax.ShapeDtypeStruct((IDX_ROWS, DIM), jnp.int32),
  )(src2d, et2d)


def _combine_body(act, p0_ref, p1_ref, hsl_ref, out_ref):
  out_ref[...] = act(p0_ref[...] + p1_ref[...] + hsl_ref[...])


def _combine(p0, p1, hsl, act):
  return pl.pallas_call(
      functools.partial(_combine_body, act),
      grid=(N_GRID,),
      in_specs=[pl.BlockSpec((N_BLK, DIM), lambda n: (n, 0))] * 3,
      out_specs=pl.BlockSpec((N_BLK, DIM), lambda n: (n, 0)),
      out_shape=jax.ShapeDtypeStruct((N_NODES, DIM), jnp.float32),
  )(p0, p1, hsl)


def _sc_body(hw_hbm, gidx_hbm, dst_hbm, out_hbm,
             gidx_v, dst_v, rows_v, agg_sh, sem):
  c = lax.axis_index("c")
  s = lax.axis_index("s")
  wid = s * NC + c

  zero = jnp.zeros((16,), jnp.float32)

  @pl.loop(0, DIM)
  def _(i):
    for j in range(DIM // 16):
      rows_v[i, pl.ds(j * 16, 16)] = zero

  for k in range(ROWS_PER_TILE // DIM):
    pltpu.sync_copy(rows_v, agg_sh.at[pl.ds(s * ROWS_PER_TILE + k * DIM, DIM)])
  plsc.subcore_barrier()

  base = wid * ROWS_PER_W
  pltpu.sync_copy(gidx_hbm.at[pl.ds(base, ROWS_PER_W)], gidx_v)
  pltpu.sync_copy(dst_hbm.at[pl.ds(base, ROWS_PER_W)], dst_v)

  @pl.loop(0, ROWS_PER_W)
  def _(j):
    pltpu.async_copy(hw_hbm.at[gidx_v.at[j]], rows_v, sem).wait()
    pltpu.sync_copy(rows_v, agg_sh.at[dst_v.at[j]], add=True)

  plsc.subcore_barrier()

  pltpu.sync_copy(agg_sh.at[pl.ds(s * ROWS_PER_TILE, ROWS_PER_TILE)],
                  out_hbm.at[c, pl.ds(s * ROWS_PER_TILE, ROWS_PER_TILE)])


def _sc_edge_agg(hw2d, gidx2d, dst2d):
  mesh = plsc.VectorSubcoreMesh(core_axis_name="c", subcore_axis_name="s",
                                num_cores=NC, num_subcores=NS)
  fn = pl.kernel(
      _sc_body,
      out_type=jax.ShapeDtypeStruct((NC, AGG_ROWS, DIM), jnp.float32),
      mesh=mesh,
      scratch_types=[
          pltpu.VMEM((ROWS_PER_W, DIM), jnp.int32),
          pltpu.VMEM((ROWS_PER_W, DIM), jnp.int32),
          pltpu.VMEM((DIM, DIM), jnp.float32),
          pltpu.VMEM_SHARED((AGG_ROWS, DIM), jnp.float32),
          pltpu.SemaphoreType.DMA,
      ],
  )
  return fn(hw2d, gidx2d, dst2d)


def kernel(node_emb, edge_index, etypes, basis1, wcomp1, loop1, bias1,
           basis2, wcomp2, loop2, bias2):
  src = edge_index[0].astype(jnp.int32)
  dst = edge_index[1].astype(jnp.int32)
  et = etypes.astype(jnp.int32)

  pad = E_PAD - N_EDGES
  src2d = jnp.pad(src, (0, pad)).reshape(IDX_ROWS, DIM)
  et2d = jnp.pad(et, (0, pad)).reshape(IDX_ROWS, DIM)
  dst2d = jnp.pad(dst, (0, pad), constant_values=N_NODES).reshape(
      IDX_ROWS, DIM)

  gidx2d = _gidx(src2d, et2d)

  def layer(h, basis, wcomp, loop_w, bias, act):
    hw, hsl = _dense_pre(h, basis, wcomp, loop_w, bias.reshape(1, DIM))
    parts = _sc_edge_agg(hw.reshape(NUM_REL * N_NODES, DIM), gidx2d, dst2d)
    return _combine(parts[0, :N_NODES], parts[1, :N_NODES], hsl, act)

  h1 = layer(node_emb, basis1, wcomp1, loop1, bias1, jnp.tanh)
  return layer(h1, basis2, wcomp2, loop2, bias2, jax.nn.relu)


# speedup vs baseline: 9.1767x; 1.0149x over previous
import functools

import jax
import jax.numpy as jnp
from jax import lax
from jax.experimental import pallas as pl
from jax.experimental.pallas import tpu as pltpu
from jax.experimental.pallas import tpu_sc as plsc

N_NODES = 10000
N_EDGES = 320000
NUM_REL = 8
DIM = 128

NC = 2
NS = 16
NW = NC * NS

E_PAD = 327680
IDX_ROWS = E_PAD // DIM
ROWS_PER_W = IDX_ROWS // NW
AGG_ROWS = 10240
ROWS_PER_TILE = AGG_ROWS // NS

N_BLK = 1000
N_GRID = N_NODES // N_BLK


def _dense_pre_body(h_ref, basis_ref, wcomp_ref, loop_ref, bias_ref,
                    hw_ref, hsl_ref):
  r = pl.program_id(1)
  wc = wcomp_ref[pl.ds(r, 1), :]
  w = wc[:, 0:1] * basis_ref[0] + wc[:, 1:2] * basis_ref[1]
  hw_ref[0] = jnp.dot(h_ref[...], w, preferred_element_type=jnp.float32,
                      precision=lax.Precision.HIGHEST)

  @pl.when(r == 0)
  def _():
    hsl_ref[...] = (
        jnp.dot(h_ref[...], loop_ref[...], preferred_element_type=jnp.float32,
                precision=lax.Precision.HIGHEST)
        + bias_ref[...])


def _dense_pre(h, basis, wcomp, loop_w, bias2d):
  return pl.pallas_call(
      _dense_pre_body,
      grid=(N_GRID, NUM_REL),
      in_specs=[
          pl.BlockSpec((N_BLK, DIM), lambda n, r: (n, 0)),
          pl.BlockSpec((2, DIM, DIM), lambda n, r: (0, 0, 0)),
          pl.BlockSpec((NUM_REL, 2), lambda n, r: (0, 0)),
          pl.BlockSpec((DIM, DIM), lambda n, r: (0, 0)),
          pl.BlockSpec((1, DIM), lambda n, r: (0, 0)),
      ],
      out_specs=[
          pl.BlockSpec((1, N_BLK, DIM), lambda n, r: (r, n, 0)),
          pl.BlockSpec((N_BLK, DIM), lambda n, r: (n, 0)),
      ],
      out_shape=[
          jax.ShapeDtypeStruct((NUM_REL, N_NODES, DIM), jnp.float32),
          jax.ShapeDtypeStruct((N_NODES, DIM), jnp.float32),
      ],
  )(h, basis, wcomp, loop_w, bias2d)


def _gidx_body(src_ref, et_ref, out_ref):
  out_ref[...] = et_ref[...] * N_NODES + src_ref[...]


def _gidx(src2d, et2d):
  blk = IDX_ROWS // 10
  return pl.pallas_call(
      _gidx_body,
      grid=(10,),
      in_specs=[
          pl.BlockSpec((blk, DIM), lambda i: (i, 0)),
          pl.BlockSpec((blk, DIM), lambda i: (i, 0)),
      ],
      out_specs=pl.BlockSpec((blk, DIM), lambda i: (i, 0)),
      out_shape=jax.ShapeDtypeStruct((IDX_ROWS, DIM), jnp.int32),
  )(src2d, et2d)


def _combine_body(act, p0_ref, p1_ref, hsl_ref, out_ref):
  out_ref[...] = act(p0_ref[...] + p1_ref[...] + hsl_ref[...])


def _combine(p0, p1, hsl, act):
  return pl.pallas_call(
      functools.partial(_combine_body, act),
      grid=(N_GRID,),
      in_specs=[pl.BlockSpec((N_BLK, DIM), lambda n: (n, 0))] * 3,
      out_specs=pl.BlockSpec((N_BLK, DIM), lambda n: (n, 0)),
      out_shape=jax.ShapeDtypeStruct((N_NODES, DIM), jnp.float32),
  )(p0, p1, hsl)


MCHUNK = 16
N_MCHUNK = ROWS_PER_W // MCHUNK


def _sc_body(hw_hbm, gidx_hbm, dst_hbm, out_hbm,
             mbufs, rbufs, msem, gsems, agg_sh):
  c = lax.axis_index("c")
  s = lax.axis_index("s")
  wid = s * NC + c
  base = wid * ROWS_PER_W

  zero = jnp.zeros((16,), jnp.float32)

  @pl.loop(0, DIM)
  def _(i):
    for j in range(DIM // 16):
      rbufs[0][i, pl.ds(j * 16, 16)] = zero

  for k in range(ROWS_PER_TILE // DIM):
    pltpu.sync_copy(rbufs[0],
                    agg_sh.at[pl.ds(s * ROWS_PER_TILE + k * DIM, DIM)])
  plsc.subcore_barrier()

  def meta_start(i):
    gv, dv = mbufs[i % 2]
    sl = pl.ds(base + i * MCHUNK, MCHUNK)
    pltpu.async_copy(gidx_hbm.at[sl], gv, msem)
    pltpu.async_copy(dst_hbm.at[sl], dv, msem)

  def meta_wait(i):
    gv, dv = mbufs[i % 2]
    sl = pl.ds(base + i * MCHUNK, MCHUNK)
    pltpu.make_async_copy(gidx_hbm.at[sl], gv, msem).wait()
    pltpu.make_async_copy(dst_hbm.at[sl], dv, msem).wait()

  meta_start(0)
  for i in range(N_MCHUNK):
    gv, dv = mbufs[i % 2]
    meta_wait(i)
    if i + 1 < N_MCHUNK:
      meta_start(i + 1)
    pltpu.async_copy(hw_hbm.at[gv.at[0]], rbufs[0], gsems[0])
    pltpu.async_copy(hw_hbm.at[gv.at[1]], rbufs[1], gsems[1])

    @pl.loop(0, MCHUNK, step=2)
    def _(b):
      for p in range(2):
        r = b + p
        pltpu.make_async_copy(hw_hbm.at[gv.at[r]], rbufs[p], gsems[p]).wait()
        pltpu.sync_copy(rbufs[p], agg_sh.at[dv.at[r]], add=True)

        @pl.when(r + 2 < MCHUNK)
        def _():
          pltpu.async_copy(hw_hbm.at[gv.at[r + 2]], rbufs[p], gsems[p])

  plsc.subcore_barrier()

  pltpu.sync_copy(agg_sh.at[pl.ds(s * ROWS_PER_TILE, ROWS_PER_TILE)],
                  out_hbm.at[c, pl.ds(s * ROWS_PER_TILE, ROWS_PER_TILE)])


def _sc_edge_agg(hw2d, gidx2d, dst2d):
  mesh = plsc.VectorSubcoreMesh(core_axis_name="c", subcore_axis_name="s",
                                num_cores=NC, num_subcores=NS)
  fn = pl.kernel(
      _sc_body,
      out_type=jax.ShapeDtypeStruct((NC, AGG_ROWS, DIM), jnp.float32),
      mesh=mesh,
      scratch_types=[
          [[pltpu.VMEM((MCHUNK, DIM), jnp.int32)] * 2] * 2,
          [pltpu.VMEM((DIM, DIM), jnp.float32)] * 2,
          pltpu.SemaphoreType.DMA,
          [pltpu.SemaphoreType.DMA] * 2,
          pltpu.VMEM_SHARED((AGG_ROWS, DIM), jnp.float32),
      ],
  )
  return fn(hw2d, gidx2d, dst2d)


def kernel(node_emb, edge_index, etypes, basis1, wcomp1, loop1, bias1,
           basis2, wcomp2, loop2, bias2):
  src = edge_index[0].astype(jnp.int32)
  dst = edge_index[1].astype(jnp.int32)
  et = etypes.astype(jnp.int32)

  pad = E_PAD - N_EDGES
  src2d = jnp.pad(src, (0, pad)).reshape(IDX_ROWS, DIM)
  et2d = jnp.pad(et, (0, pad)).reshape(IDX_ROWS, DIM)
  dst2d = jnp.pad(dst, (0, pad), constant_values=N_NODES).reshape(
      IDX_ROWS, DIM)

  gidx2d = _gidx(src2d, et2d)

  def layer(h, basis, wcomp, loop_w, bias, act):
    hw, hsl = _dense_pre(h, basis, wcomp, loop_w, bias.reshape(1, DIM))
    parts = _sc_edge_agg(hw.reshape(NUM_REL * N_NODES, DIM), gidx2d, dst2d)
    return _combine(parts[0, :N_NODES], parts[1, :N_NODES], hsl, act)

  h1 = layer(node_emb, basis1, wcomp1, loop1, bias1, jnp.tanh)
  return layer(h1, basis2, wcomp2, loop2, bias2, jax.nn.relu)


# speedup vs baseline: 24.0487x; 2.6206x over previous
import functools

import jax
import jax.numpy as jnp
from jax import lax
from jax.experimental import pallas as pl
from jax.experimental.pallas import tpu as pltpu
from jax.experimental.pallas import tpu_sc as plsc

N_NODES = 10000
N_EDGES = 320000
NUM_REL = 8
DIM = 128

NC = 2
NS = 16
NW = NC * NS

E_PAD = 327680
IDX_ROWS = E_PAD // DIM
ROWS_PER_W = IDX_ROWS // NW
AGG_ROWS = 10240
ROWS_PER_TILE = AGG_ROWS // NS

N_BLK = 1000
N_GRID = N_NODES // N_BLK


def _dense_pre_body(h_ref, basis_ref, wcomp_ref, loop_ref, bias_ref,
                    hw_ref, hsl_ref):
  r = pl.program_id(1)
  wc = wcomp_ref[pl.ds(r, 1), :]
  w = wc[:, 0:1] * basis_ref[0] + wc[:, 1:2] * basis_ref[1]
  hw_ref[0] = jnp.dot(h_ref[...], w, preferred_element_type=jnp.float32,
                      precision=lax.Precision.HIGHEST)

  @pl.when(r == 0)
  def _():
    hsl_ref[...] = (
        jnp.dot(h_ref[...], loop_ref[...], preferred_element_type=jnp.float32,
                precision=lax.Precision.HIGHEST)
        + bias_ref[...])


def _dense_pre(h, basis, wcomp, loop_w, bias2d):
  return pl.pallas_call(
      _dense_pre_body,
      grid=(N_GRID, NUM_REL),
      in_specs=[
          pl.BlockSpec((N_BLK, DIM), lambda n, r: (n, 0)),
          pl.BlockSpec((2, DIM, DIM), lambda n, r: (0, 0, 0)),
          pl.BlockSpec((NUM_REL, 2), lambda n, r: (0, 0)),
          pl.BlockSpec((DIM, DIM), lambda n, r: (0, 0)),
          pl.BlockSpec((1, DIM), lambda n, r: (0, 0)),
      ],
      out_specs=[
          pl.BlockSpec((1, N_BLK, DIM), lambda n, r: (r, n, 0)),
          pl.BlockSpec((N_BLK, DIM), lambda n, r: (n, 0)),
      ],
      out_shape=[
          jax.ShapeDtypeStruct((NUM_REL, N_NODES, DIM), jnp.float32),
          jax.ShapeDtypeStruct((N_NODES, DIM), jnp.float32),
      ],
  )(h, basis, wcomp, loop_w, bias2d)


def _gidx_body(src_ref, et_ref, out_ref):
  out_ref[...] = et_ref[...] * N_NODES + src_ref[...]


def _gidx(src2d, et2d):
  blk = IDX_ROWS // 10
  return pl.pallas_call(
      _gidx_body,
      grid=(10,),
      in_specs=[
          pl.BlockSpec((blk, DIM), lambda i: (i, 0)),
          pl.BlockSpec((blk, DIM), lambda i: (i, 0)),
      ],
      out_specs=pl.BlockSpec((blk, DIM), lambda i: (i, 0)),
      out_shape=jax.ShapeDtypeStruct((IDX_ROWS, DIM), jnp.int32),
  )(src2d, et2d)


def _combine_body(act, p0_ref, p1_ref, hsl_ref, out_ref):
  out_ref[...] = act(p0_ref[...] + p1_ref[...] + hsl_ref[...])


def _combine(p0, p1, hsl, act):
  return pl.pallas_call(
      functools.partial(_combine_body, act),
      grid=(N_GRID,),
      in_specs=[pl.BlockSpec((N_BLK, DIM), lambda n: (n, 0))] * 3,
      out_specs=pl.BlockSpec((N_BLK, DIM), lambda n: (n, 0)),
      out_shape=jax.ShapeDtypeStruct((N_NODES, DIM), jnp.float32),
  )(p0, p1, hsl)


MCHUNK = 16
N_MCHUNK = ROWS_PER_W // MCHUNK


def _sc_body(hw_hbm, gidx_hbm, dst_hbm, out_hbm,
             mbufs, rbufs, msem, gsems, agg_sh):
  c = lax.axis_index("c")
  s = lax.axis_index("s")
  wid = s * NC + c
  base = wid * ROWS_PER_W

  zero = jnp.zeros((16,), jnp.float32)

  @pl.loop(0, DIM)
  def _(i):
    for j in range(DIM // 16):
      rbufs[0][i, pl.ds(j * 16, 16)] = zero

  for k in range(ROWS_PER_TILE // DIM):
    pltpu.sync_copy(rbufs[0],
                    agg_sh.at[pl.ds(s * ROWS_PER_TILE + k * DIM, DIM)])
  plsc.subcore_barrier()

  def meta_start(i):
    gv, dv = mbufs[i % 2]
    sl = pl.ds(base + i * MCHUNK, MCHUNK)
    pltpu.async_copy(gidx_hbm.at[sl], gv, msem)
    pltpu.async_copy(dst_hbm.at[sl], dv, msem)

  def meta_wait(i):
    gv, dv = mbufs[i % 2]
    sl = pl.ds(base + i * MCHUNK, MCHUNK)
    pltpu.make_async_copy(gidx_hbm.at[sl], gv, msem).wait()
    pltpu.make_async_copy(dst_hbm.at[sl], dv, msem).wait()

  meta_start(0)
  for i in range(N_MCHUNK):
    gv, dv = mbufs[i % 2]
    meta_wait(i)
    if i + 1 < N_MCHUNK:
      meta_start(i + 1)
    pltpu.async_copy(hw_hbm.at[gv.at[0]], rbufs[0], gsems[0])
    pltpu.async_copy(hw_hbm.at[gv.at[1]], rbufs[1], gsems[1])

    @pl.loop(0, MCHUNK, step=2)
    def _(b):
      for p in range(2):
        r = b + p
        pltpu.make_async_copy(hw_hbm.at[gv.at[r]], rbufs[p], gsems[p]).wait()
        pltpu.sync_copy(rbufs[p], agg_sh.at[dv.at[r]], add=True)

        @pl.when(r + 2 < MCHUNK)
        def _():
          pltpu.async_copy(hw_hbm.at[gv.at[r + 2]], rbufs[p], gsems[p])

  plsc.subcore_barrier()

  pltpu.sync_copy(agg_sh.at[pl.ds(s * ROWS_PER_TILE, ROWS_PER_TILE)],
                  out_hbm.at[c, pl.ds(s * ROWS_PER_TILE, ROWS_PER_TILE)])


def _sc_edge_agg(hw2d, gidx2d, dst2d):
  mesh = plsc.VectorSubcoreMesh(core_axis_name="c", subcore_axis_name="s",
                                num_cores=NC, num_subcores=NS)
  fn = pl.kernel(
      _sc_body,
      out_type=jax.ShapeDtypeStruct((NC, AGG_ROWS, DIM), jnp.float32),
      mesh=mesh,
      scratch_types=[
          [[pltpu.VMEM((MCHUNK, DIM), jnp.int32)] * 2] * 2,
          [pltpu.VMEM((DIM, DIM), jnp.float32)] * 2,
          pltpu.SemaphoreType.DMA,
          [pltpu.SemaphoreType.DMA] * 2,
          pltpu.VMEM_SHARED((AGG_ROWS, DIM), jnp.float32),
      ],
  )
  return fn(hw2d, gidx2d, dst2d)


def kernel(node_emb, edge_index, etypes, basis1, wcomp1, loop1, bias1,
           basis2, wcomp2, loop2, bias2):
  src = edge_index[0].astype(jnp.int32)
  dst = edge_index[1].astype(jnp.int32)
  et = etypes.astype(jnp.int32)

  pad = E_PAD - N_EDGES
  pad_iota = jnp.arange(pad, dtype=jnp.int32)
  src2d = jnp.concatenate([src, pad_iota % N_NODES]).reshape(IDX_ROWS, DIM)
  et2d = jnp.pad(et, (0, pad)).reshape(IDX_ROWS, DIM)
  dst2d = jnp.concatenate(
      [dst, N_NODES + pad_iota % (AGG_ROWS - N_NODES)]).reshape(IDX_ROWS, DIM)

  gidx2d = _gidx(src2d, et2d)

  def layer(h, basis, wcomp, loop_w, bias, act):
    hw, hsl = _dense_pre(h, basis, wcomp, loop_w, bias.reshape(1, DIM))
    parts = _sc_edge_agg(hw.reshape(NUM_REL * N_NODES, DIM), gidx2d, dst2d)
    return _combine(parts[0, :N_NODES], parts[1, :N_NODES], hsl, act)

  h1 = layer(node_emb, basis1, wcomp1, loop1, bias1, jnp.tanh)
  return layer(h1, basis2, wcomp2, loop2, bias2, jax.nn.relu)
